# baseline (speedup 1.0000x reference)
import jax
import jax.numpy as jnp
from jax import lax
from jax.experimental import pallas as pl
from jax.experimental.pallas import tpu as pltpu
from jax.experimental.pallas import tpu_sc as plsc

N = 10000
E = 320000
D = 128
DH = D // 2
NC = 2
NS = 16
N_PAD = 10240
ROWS_N = N_PAD // NS
CH = 128
E_ROWS = 2560
E_PAD = E_ROWS * CH
ROWS_E_AGG = E_ROWS // NS
ROWS_E_DEG = E_ROWS // (NC * NS)
DEG_W = 8

_sc_mesh = plsc.VectorSubcoreMesh(core_axis_name="c", subcore_axis_name="s")


def _deg_body(dstr, ones_hbm, zeros_hbm, out, ones_v, dst_v, deg_sh):
    c = lax.axis_index("c")
    s = lax.axis_index("s")
    pltpu.sync_copy(zeros_hbm, deg_sh.at[pl.ds(s * ROWS_N, ROWS_N)])
    pltpu.sync_copy(ones_hbm, ones_v)
    wid = c * NS + s
    pltpu.sync_copy(dstr.at[pl.ds(wid * ROWS_E_DEG, ROWS_E_DEG)], dst_v)
    plsc.subcore_barrier()

    def body(j, carry):
        pltpu.sync_copy(ones_v, deg_sh.at[dst_v.at[j]], add=True)
        return carry

    lax.fori_loop(0, ROWS_E_DEG, body, 0)
    plsc.subcore_barrier()
    pltpu.sync_copy(deg_sh.at[pl.ds(s * ROWS_N, ROWS_N)],
                    out.at[c, pl.ds(s * ROWS_N, ROWS_N)])


_deg_kernel = pl.kernel(
    _deg_body,
    out_type=jax.ShapeDtypeStruct((NC, N_PAD, DEG_W), jnp.float32),
    mesh=_sc_mesh,
    scratch_types=[
        pltpu.VMEM((CH, DEG_W), jnp.float32),
        pltpu.VMEM((ROWS_E_DEG, CH), jnp.int32),
        pltpu.VMEM_SHARED((N_PAD, DEG_W), jnp.float32),
    ],
    compiler_params=pltpu.CompilerParams(use_tc_tiling_on_sc=False),
)


NBUF = 4


def _agg_body(y_hbm, srcr2, dstr, zeros_hbm, out, *scr):
    src_v, dst_v = scr[0], scr[1]
    bufs = scr[2:2 + NBUF]
    agg_sh = scr[2 + NBUF]
    gsems = scr[3 + NBUF:3 + 2 * NBUF]
    ssems = scr[3 + 2 * NBUF:]
    c = lax.axis_index("c")
    s = lax.axis_index("s")
    pltpu.sync_copy(zeros_hbm, agg_sh.at[pl.ds(s * ROWS_N, ROWS_N)])
    pltpu.sync_copy(srcr2.at[c, pl.ds(s * ROWS_E_AGG, ROWS_E_AGG)], src_v)
    pltpu.sync_copy(dstr.at[pl.ds(s * ROWS_E_AGG, ROWS_E_AGG)], dst_v)

    def _wait(sem, b):
        pltpu.make_async_copy(y_hbm.at[pl.ds(0, CH)], bufs[b], sem).wait()

    for b in range(NBUF):
        pltpu.async_copy(y_hbm.at[src_v.at[b]], bufs[b], gsems[b])
    plsc.subcore_barrier()

    def body(i, carry):
        j0 = i * NBUF
        for b in range(NBUF):
            _wait(gsems[b], b)
            pltpu.async_copy(bufs[b], agg_sh.at[dst_v.at[j0 + b]],
                             ssems[b], add=True)
        for b in range(NBUF):
            _wait(ssems[b], b)
            pltpu.async_copy(y_hbm.at[src_v.at[j0 + NBUF + b]],
                             bufs[b], gsems[b])
        return carry

    lax.fori_loop(0, ROWS_E_AGG // NBUF - 1, body, 0)
    j0 = ROWS_E_AGG - NBUF
    for b in range(NBUF):
        _wait(gsems[b], b)
        pltpu.async_copy(bufs[b], agg_sh.at[dst_v.at[j0 + b]],
                         ssems[b], add=True)
    for b in range(NBUF):
        _wait(ssems[b], b)

    plsc.subcore_barrier()
    pltpu.sync_copy(agg_sh.at[pl.ds(s * ROWS_N, ROWS_N)],
                    out.at[c, pl.ds(s * ROWS_N, ROWS_N)])


_agg_kernel = pl.kernel(
    _agg_body,
    out_type=jax.ShapeDtypeStruct((NC, N_PAD, DH), jnp.float32),
    mesh=_sc_mesh,
    scratch_types=(
        [pltpu.VMEM((ROWS_E_AGG, CH), jnp.int32),
         pltpu.VMEM((ROWS_E_AGG, CH), jnp.int32)]
        + [pltpu.VMEM((CH, DH), jnp.float32)] * NBUF
        + [pltpu.VMEM_SHARED((N_PAD, DH), jnp.float32)]
        + [pltpu.SemaphoreType.DMA] * (2 * NBUF)
    ),
    compiler_params=pltpu.CompilerParams(use_tc_tiling_on_sc=False),
)

BN = 1024
BN_E = 1000


def _rsqrt_deg(degp):
    return lax.rsqrt(degp[0, :, 0:1] + degp[1, :, 0:1] + 1.0)


def _y1_body(x_ref, w_ref, degp_ref, y_ref):
    d = _rsqrt_deg(degp_ref[...])
    xw = jnp.dot(x_ref[...], w_ref[...], preferred_element_type=jnp.float32)
    y = xw * d
    y_ref[0] = y[:, :DH]
    y_ref[1] = y[:, DH:]


_y1_call = pl.pallas_call(
    _y1_body,
    grid=(N_PAD // BN,),
    in_specs=[
        pl.BlockSpec((BN, D), lambda i: (i, 0)),
        pl.BlockSpec((D, D), lambda i: (0, 0)),
        pl.BlockSpec((NC, BN, DEG_W), lambda i: (0, i, 0)),
    ],
    out_specs=pl.BlockSpec((NC, BN, DH), lambda i: (0, i, 0)),
    out_shape=jax.ShapeDtypeStruct((NC, N_PAD, DH), jnp.float32),
)


def _mid_body(agg_ref, y_ref, degp_ref, b_ref, w_ref, o_ref):
    d = _rsqrt_deg(degp_ref[...])
    g = jnp.concatenate(
        [agg_ref[0] + y_ref[0], agg_ref[1] + y_ref[1]], axis=1)
    h = jnp.maximum(g * d + b_ref[...], 0.0)
    y2 = jnp.dot(h, w_ref[...], preferred_element_type=jnp.float32) * d
    o_ref[0] = y2[:, :DH]
    o_ref[1] = y2[:, DH:]


_mid_call = pl.pallas_call(
    _mid_body,
    grid=(N_PAD // BN,),
    in_specs=[
        pl.BlockSpec((NC, BN, DH), lambda i: (0, i, 0)),
        pl.BlockSpec((NC, BN, DH), lambda i: (0, i, 0)),
        pl.BlockSpec((NC, BN, DEG_W), lambda i: (0, i, 0)),
        pl.BlockSpec((1, D), lambda i: (0, 0)),
        pl.BlockSpec((D, D), lambda i: (0, 0)),
    ],
    out_specs=pl.BlockSpec((NC, BN, DH), lambda i: (0, i, 0)),
    out_shape=jax.ShapeDtypeStruct((NC, N_PAD, DH), jnp.float32),
)


def _out_body(agg_ref, y_ref, degp_ref, b_ref, o_ref):
    d = _rsqrt_deg(degp_ref[...])
    g = jnp.concatenate(
        [agg_ref[0] + y_ref[0], agg_ref[1] + y_ref[1]], axis=1)
    o_ref[...] = g * d + b_ref[...]


_out_call = pl.pallas_call(
    _out_body,
    grid=(N // BN_E,),
    in_specs=[
        pl.BlockSpec((NC, BN_E, DH), lambda i: (0, i, 0)),
        pl.BlockSpec((NC, BN_E, DH), lambda i: (0, i, 0)),
        pl.BlockSpec((NC, BN_E, DEG_W), lambda i: (0, i, 0)),
        pl.BlockSpec((1, D), lambda i: (0, 0)),
    ],
    out_specs=pl.BlockSpec((BN_E, D), lambda i: (i, 0)),
    out_shape=jax.ShapeDtypeStruct((N, D), jnp.float32),
)


def kernel(x, edge_index, W1, b1, W2, b2):
    src = edge_index[0]
    dst = edge_index[1]
    padi = jnp.full((E_PAD - E,), N, dtype=jnp.int32)
    srcr = jnp.concatenate([src, padi]).reshape(E_ROWS, CH)
    srcr2 = jnp.stack([srcr, srcr + N_PAD])
    dstr = jnp.concatenate([dst, padi]).reshape(E_ROWS, CH)
    x_pad = jnp.zeros((N_PAD, D), x.dtype).at[:N].set(x)
    zeros_deg = jnp.zeros((ROWS_N, DEG_W), jnp.float32)
    ones_deg = jnp.ones((CH, DEG_W), jnp.float32)
    zeros_agg = jnp.zeros((ROWS_N, DH), jnp.float32)
    b1r = b1.reshape(1, D)
    b2r = b2.reshape(1, D)

    degp = _deg_kernel(dstr, ones_deg, zeros_deg)
    y1 = _y1_call(x_pad, W1, degp)
    agg1 = _agg_kernel(y1.reshape(NC * N_PAD, DH), srcr2, dstr, zeros_agg)
    y2 = _mid_call(agg1, y1, degp, b1r, W2)
    agg2 = _agg_kernel(y2.reshape(NC * N_PAD, DH), srcr2, dstr, zeros_agg)
    out = _out_call(agg2, y2, degp, b2r)
    return out


# speedup vs baseline: 1.0301x; 1.0301x over previous
import jax
import jax.numpy as jnp
from jax import lax
from jax.experimental import pallas as pl
from jax.experimental.pallas import tpu as pltpu
from jax.experimental.pallas import tpu_sc as plsc

N = 10000
E = 320000
D = 128
DH = D // 2
NC = 2
NS = 16
N_PAD = 10240
ROWS_N = N_PAD // NS
CH = 128
E_ROWS = 2560
E_PAD = E_ROWS * CH
ROWS_E_AGG = E_ROWS // NS
ROWS_E_DEG = E_ROWS // (NC * NS)
DEG_W = 8

_sc_mesh = plsc.VectorSubcoreMesh(core_axis_name="c", subcore_axis_name="s")


def _deg_body(dstr, ones_hbm, zeros_hbm, out, ones_v, dst_v, deg_sh):
    c = lax.axis_index("c")
    s = lax.axis_index("s")
    pltpu.sync_copy(zeros_hbm, deg_sh.at[pl.ds(s * ROWS_N, ROWS_N)])
    pltpu.sync_copy(ones_hbm, ones_v)
    wid = c * NS + s
    pltpu.sync_copy(dstr.at[pl.ds(wid * ROWS_E_DEG, ROWS_E_DEG)], dst_v)
    plsc.subcore_barrier()

    def body(j, carry):
        pltpu.sync_copy(ones_v, deg_sh.at[dst_v.at[j]], add=True)
        return carry

    lax.fori_loop(0, ROWS_E_DEG, body, 0)
    plsc.subcore_barrier()
    pltpu.sync_copy(deg_sh.at[pl.ds(s * ROWS_N, ROWS_N)],
                    out.at[c, pl.ds(s * ROWS_N, ROWS_N)])


_deg_kernel = pl.kernel(
    _deg_body,
    out_type=jax.ShapeDtypeStruct((NC, N_PAD, DEG_W), jnp.float32),
    mesh=_sc_mesh,
    scratch_types=[
        pltpu.VMEM((CH, DEG_W), jnp.float32),
        pltpu.VMEM((ROWS_E_DEG, CH), jnp.int32),
        pltpu.VMEM_SHARED((N_PAD, DEG_W), jnp.float32),
    ],
    compiler_params=pltpu.CompilerParams(use_tc_tiling_on_sc=False),
)


NBUF = 4


def _agg_body(y_hbm, srcr2, dstr, zeros_hbm, out, *scr):
    src_v, dst_v = scr[0], scr[1]
    bufs = scr[2:2 + NBUF]
    agg_sh = scr[2 + NBUF]
    gsems = scr[3 + NBUF:]
    c = lax.axis_index("c")
    s = lax.axis_index("s")
    pltpu.sync_copy(zeros_hbm, agg_sh.at[pl.ds(s * ROWS_N, ROWS_N)])
    pltpu.sync_copy(srcr2.at[c, pl.ds(s * ROWS_E_AGG, ROWS_E_AGG)], src_v)
    pltpu.sync_copy(dstr.at[pl.ds(s * ROWS_E_AGG, ROWS_E_AGG)], dst_v)

    def _wait(sem, b):
        pltpu.make_async_copy(y_hbm.at[pl.ds(0, CH)], bufs[b], sem).wait()

    for b in range(NBUF):
        pltpu.async_copy(y_hbm.at[src_v.at[b]], bufs[b], gsems[b])
    plsc.subcore_barrier()

    def body(i, carry):
        j0 = i * NBUF
        for b in range(NBUF):
            j = j0 + b
            _wait(gsems[b], b)
            pltpu.sync_copy(bufs[b], agg_sh.at[dst_v.at[j]], add=True)
            pltpu.async_copy(y_hbm.at[src_v.at[j + NBUF]], bufs[b], gsems[b])
        return carry

    lax.fori_loop(0, ROWS_E_AGG // NBUF - 1, body, 0)
    for b in range(NBUF):
        j = ROWS_E_AGG - NBUF + b
        _wait(gsems[b], b)
        pltpu.sync_copy(bufs[b], agg_sh.at[dst_v.at[j]], add=True)

    plsc.subcore_barrier()
    pltpu.sync_copy(agg_sh.at[pl.ds(s * ROWS_N, ROWS_N)],
                    out.at[c, pl.ds(s * ROWS_N, ROWS_N)])


_agg_kernel = pl.kernel(
    _agg_body,
    out_type=jax.ShapeDtypeStruct((NC, N_PAD, DH), jnp.float32),
    mesh=_sc_mesh,
    scratch_types=(
        [pltpu.VMEM((ROWS_E_AGG, CH), jnp.int32),
         pltpu.VMEM((ROWS_E_AGG, CH), jnp.int32)]
        + [pltpu.VMEM((CH, DH), jnp.float32)] * NBUF
        + [pltpu.VMEM_SHARED((N_PAD, DH), jnp.float32)]
        + [pltpu.SemaphoreType.DMA] * NBUF
    ),
    compiler_params=pltpu.CompilerParams(use_tc_tiling_on_sc=False),
)

BN = 1024
BN_E = 1000


def _rsqrt_deg(degp):
    return lax.rsqrt(degp[0, :, 0:1] + degp[1, :, 0:1] + 1.0)


def _xw_body(x_ref, w_ref, xw_ref):
    xw_ref[...] = jnp.dot(x_ref[...], w_ref[...],
                          preferred_element_type=jnp.float32)


_xw_call = pl.pallas_call(
    _xw_body,
    grid=(N_PAD // BN,),
    in_specs=[
        pl.BlockSpec((BN, D), lambda i: (i, 0)),
        pl.BlockSpec((D, D), lambda i: (0, 0)),
    ],
    out_specs=pl.BlockSpec((BN, D), lambda i: (i, 0)),
    out_shape=jax.ShapeDtypeStruct((N_PAD, D), jnp.float32),
)


def _scale_body(xw_ref, degp_ref, y_ref):
    d = _rsqrt_deg(degp_ref[...])
    y = xw_ref[...] * d
    y_ref[0] = y[:, :DH]
    y_ref[1] = y[:, DH:]


_scale_call = pl.pallas_call(
    _scale_body,
    grid=(N_PAD // BN,),
    in_specs=[
        pl.BlockSpec((BN, D), lambda i: (i, 0)),
        pl.BlockSpec((NC, BN, DEG_W), lambda i: (0, i, 0)),
    ],
    out_specs=pl.BlockSpec((NC, BN, DH), lambda i: (0, i, 0)),
    out_shape=jax.ShapeDtypeStruct((NC, N_PAD, DH), jnp.float32),
)


def _mid_body(agg_ref, y_ref, degp_ref, b_ref, w_ref, o_ref):
    d = _rsqrt_deg(degp_ref[...])
    g = jnp.concatenate(
        [agg_ref[0] + y_ref[0], agg_ref[1] + y_ref[1]], axis=1)
    h = jnp.maximum(g * d + b_ref[...], 0.0)
    y2 = jnp.dot(h, w_ref[...], preferred_element_type=jnp.float32) * d
    o_ref[0] = y2[:, :DH]
    o_ref[1] = y2[:, DH:]


_mid_call = pl.pallas_call(
    _mid_body,
    grid=(N_PAD // BN,),
    in_specs=[
        pl.BlockSpec((NC, BN, DH), lambda i: (0, i, 0)),
        pl.BlockSpec((NC, BN, DH), lambda i: (0, i, 0)),
        pl.BlockSpec((NC, BN, DEG_W), lambda i: (0, i, 0)),
        pl.BlockSpec((1, D), lambda i: (0, 0)),
        pl.BlockSpec((D, D), lambda i: (0, 0)),
    ],
    out_specs=pl.BlockSpec((NC, BN, DH), lambda i: (0, i, 0)),
    out_shape=jax.ShapeDtypeStruct((NC, N_PAD, DH), jnp.float32),
)


def _out_body(agg_ref, y_ref, degp_ref, b_ref, o_ref):
    d = _rsqrt_deg(degp_ref[...])
    g = jnp.concatenate(
        [agg_ref[0] + y_ref[0], agg_ref[1] + y_ref[1]], axis=1)
    o_ref[...] = g * d + b_ref[...]


_out_call = pl.pallas_call(
    _out_body,
    grid=(N // BN_E,),
    in_specs=[
        pl.BlockSpec((NC, BN_E, DH), lambda i: (0, i, 0)),
        pl.BlockSpec((NC, BN_E, DH), lambda i: (0, i, 0)),
        pl.BlockSpec((NC, BN_E, DEG_W), lambda i: (0, i, 0)),
        pl.BlockSpec((1, D), lambda i: (0, 0)),
    ],
    out_specs=pl.BlockSpec((BN_E, D), lambda i: (i, 0)),
    out_shape=jax.ShapeDtypeStruct((N, D), jnp.float32),
)


def kernel(x, edge_index, W1, b1, W2, b2):
    src = edge_index[0]
    dst = edge_index[1]
    padi = jnp.full((E_PAD - E,), N, dtype=jnp.int32)
    srcr = jnp.concatenate([src, padi]).reshape(E_ROWS, CH)
    srcr2 = jnp.stack([srcr, srcr + N_PAD])
    dstr = jnp.concatenate([dst, padi]).reshape(E_ROWS, CH)
    x_pad = jnp.zeros((N_PAD, D), x.dtype).at[:N].set(x)
    zeros_deg = jnp.zeros((ROWS_N, DEG_W), jnp.float32)
    ones_deg = jnp.ones((CH, DEG_W), jnp.float32)
    zeros_agg = jnp.zeros((ROWS_N, DH), jnp.float32)
    b1r = b1.reshape(1, D)
    b2r = b2.reshape(1, D)

    degp = _deg_kernel(dstr, ones_deg, zeros_deg)
    xw1 = _xw_call(x_pad, W1)
    y1 = _scale_call(xw1, degp)
    agg1 = _agg_kernel(y1.reshape(NC * N_PAD, DH), srcr2, dstr, zeros_agg)
    y2 = _mid_call(agg1, y1, degp, b1r, W2)
    agg2 = _agg_kernel(y2.reshape(NC * N_PAD, DH), srcr2, dstr, zeros_agg)
    out = _out_call(agg2, y2, degp, b2r)
    return out
